# Initial kernel scaffold; baseline (speedup 1.0000x reference)
import jax
import jax.numpy as jnp
from jax.experimental import pallas as pl


def kernel(xyz, features, W0, b0, gamma0, beta0, W1, b1, gamma1, beta1, W2, b2, gamma2, beta2):
    raise NotImplementedError("write your pallas kernel here")


# speedup vs baseline: 309.0366x; 309.0366x over previous
import functools

import jax
import jax.numpy as jnp
from jax import lax
from jax.experimental import pallas as pl
from jax.experimental.pallas import tpu as pltpu
from jax.experimental.pallas import tpu_sc as plsc

B = 8
N = 4096
S = 1024
K = 32
TQ = 256
TR = 2048
QT = TR // K


def _fps_body(xt_ref, out_ref):
  X = xt_ref[0]
  Y = xt_ref[1]
  Z = xt_ref[2]
  lane_n = lax.broadcasted_iota(jnp.int32, (B, N), 1)
  lane_s = lax.broadcasted_iota(jnp.int32, (B, S), 1)

  def body(i, st):
    dists, far = st
    m = lane_n == far
    cx = jnp.sum(jnp.where(m, X, 0.0), axis=1, keepdims=True)
    cy = jnp.sum(jnp.where(m, Y, 0.0), axis=1, keepdims=True)
    cz = jnp.sum(jnp.where(m, Z, 0.0), axis=1, keepdims=True)
    sel = lane_s == i
    out_ref[0] = jnp.where(sel, cx, out_ref[0])
    out_ref[1] = jnp.where(sel, cy, out_ref[1])
    out_ref[2] = jnp.where(sel, cz, out_ref[2])
    dx = X - cx
    dy = Y - cy
    dz = Z - cz
    d = dx * dx + dy * dy + dz * dz
    dists = jnp.minimum(dists, d)
    far = jnp.argmax(dists, axis=1).astype(jnp.int32)[:, None]
    return dists, far

  lax.fori_loop(
      0, S, body,
      (jnp.full((B, N), 1e10, jnp.float32), jnp.zeros((B, 1), jnp.int32)))


def _fps(xt):
  return pl.pallas_call(
      _fps_body,
      out_shape=jax.ShapeDtypeStruct((3, B, S), jnp.float32),
  )(xt)


def _knn_body(q_ref, xp_ref, out_ref):
  b = pl.program_id(0)
  qt = q_ref[0]
  xp = xp_ref[0]
  dx = qt[:, 0:1] - xp[0:1, :]
  dy = qt[:, 1:2] - xp[1:2, :]
  dz = qt[:, 2:3] - xp[2:3, :]
  d = dx * dx + dy * dy + dz * dz
  lane = lax.broadcasted_iota(jnp.int32, (TQ, N), 1)
  base = b * N
  for j in range(K):
    am = jnp.argmin(d, axis=1).astype(jnp.int32)
    out_ref[0, :, j:j + 1] = (am + base)[:, None]
    d = jnp.where(lane == am[:, None], jnp.float32(jnp.inf), d)


def _knn(q, xp):
  return pl.pallas_call(
      _knn_body,
      grid=(B, S // TQ),
      in_specs=[
          pl.BlockSpec((1, TQ, 8), lambda b, t: (b, t, 0)),
          pl.BlockSpec((1, 8, N), lambda b, t: (b, 0, 0)),
      ],
      out_specs=pl.BlockSpec((1, TQ, K), lambda b, t: (b, t, 0)),
      out_shape=jax.ShapeDtypeStruct((B, S, K), jnp.int32),
  )(q, xp)


def _p0_body(f_ref, xp_ref, q_ref, w0f_ref, w0x_ref, b0_ref, t_ref, qp_ref):
  f = f_ref[0]
  xp = xp_ref[0]
  q = q_ref[0]
  dn = (((0,), (0,)), ((), ()))
  tf = lax.dot_general(f, w0f_ref[...], dn,
                       preferred_element_type=jnp.float32)
  tx = lax.dot_general(xp, w0x_ref[...], dn,
                       preferred_element_type=jnp.float32)
  t_ref[0] = tf + tx + b0_ref[...]
  qp_ref[0] = lax.dot_general(q, w0x_ref[...], (((1,), (0,)), ((), ())),
                              preferred_element_type=jnp.float32)


def _p0(features, xp, q, w0f, w0x, b0):
  return pl.pallas_call(
      _p0_body,
      grid=(B,),
      in_specs=[
          pl.BlockSpec((1, 128, N), lambda b: (b, 0, 0)),
          pl.BlockSpec((1, 8, N), lambda b: (b, 0, 0)),
          pl.BlockSpec((1, S, 8), lambda b: (b, 0, 0)),
          pl.BlockSpec((128, 128), lambda b: (0, 0)),
          pl.BlockSpec((8, 128), lambda b: (0, 0)),
          pl.BlockSpec((1, 128), lambda b: (0, 0)),
      ],
      out_specs=[
          pl.BlockSpec((1, N, 128), lambda b: (b, 0, 0)),
          pl.BlockSpec((1, S, 128), lambda b: (b, 0, 0)),
      ],
      out_shape=[
          jax.ShapeDtypeStruct((B, N, 128), jnp.float32),
          jax.ShapeDtypeStruct((B, S, 128), jnp.float32),
      ],
  )(features, xp, q, w0f, w0x, b0)


def _sc_gather(table, gidx):
  tot, d = B * S * K, 128
  info = plsc.get_sparse_core_info()
  nw = info.num_cores * info.num_subcores
  per_w = tot // nw
  ch = 128
  n_ch = per_w // ch
  mesh = plsc.VectorSubcoreMesh(core_axis_name="c", subcore_axis_name="s")

  @functools.partial(
      pl.kernel,
      mesh=mesh,
      out_type=jax.ShapeDtypeStruct((tot, d), jnp.float32),
      scratch_types=[
          pltpu.VMEM((ch,), jnp.int32),
          pltpu.VMEM((ch, d), jnp.float32),
          pltpu.SemaphoreType.DMA,
      ],
  )
  def gk(table_hbm, idx_hbm, out_hbm, idx_v, rows_v, sem):
    wid = lax.axis_index("s") * info.num_cores + lax.axis_index("c")
    base = wid * per_w

    def chunk(t, carry):
      off = base + t * ch
      pltpu.sync_copy(idx_hbm.at[pl.ds(off, ch)], idx_v)
      pltpu.async_copy(table_hbm.at[idx_v], rows_v, sem).wait()
      pltpu.sync_copy(rows_v, out_hbm.at[pl.ds(off, ch)])
      return carry

    lax.fori_loop(0, n_ch, chunk, 0)

  return gk(table, gidx)


def _expand_qp(qp):
  return jnp.reshape(
      jnp.broadcast_to(qp[:, None, :], (QT, K, qp.shape[-1])),
      (TR, qp.shape[-1]))


def _p1_body(g_ref, qp_ref, s1_ref, s2_ref):
  y0 = g_ref[...] - _expand_qp(qp_ref[...])
  p1 = jnp.sum(y0, axis=0, keepdims=True)
  p2 = jnp.sum(y0 * y0, axis=0, keepdims=True)

  @pl.when(pl.program_id(0) == 0)
  def _():
    s1_ref[...] = jnp.zeros_like(s1_ref)
    s2_ref[...] = jnp.zeros_like(s2_ref)

  s1_ref[...] += p1
  s2_ref[...] += p2


def _p1(gath, qp2):
  return pl.pallas_call(
      _p1_body,
      grid=(B * S * K // TR,),
      in_specs=[
          pl.BlockSpec((TR, 128), lambda t: (t, 0)),
          pl.BlockSpec((QT, 128), lambda t: (t, 0)),
      ],
      out_specs=[
          pl.BlockSpec((1, 128), lambda t: (0, 0)),
          pl.BlockSpec((1, 128), lambda t: (0, 0)),
      ],
      out_shape=[
          jax.ShapeDtypeStruct((1, 128), jnp.float32),
          jax.ShapeDtypeStruct((1, 128), jnp.float32),
      ],
  )(gath, qp2)


def _p2_body(g_ref, qp_ref, a_ref, c_ref, w_ref, y_ref, s1_ref, s2_ref):
  y0 = g_ref[...] - _expand_qp(qp_ref[...])
  h = jnp.maximum(a_ref[...] * y0 + c_ref[...], 0.0)
  y = jnp.dot(h, w_ref[...], preferred_element_type=jnp.float32)
  y_ref[...] = y
  p1 = jnp.sum(y, axis=0, keepdims=True)
  p2 = jnp.sum(y * y, axis=0, keepdims=True)

  @pl.when(pl.program_id(0) == 0)
  def _():
    s1_ref[...] = jnp.zeros_like(s1_ref)
    s2_ref[...] = jnp.zeros_like(s2_ref)

  s1_ref[...] += p1
  s2_ref[...] += p2


def _p2(gath, qp2, a0, c0, w1t):
  return pl.pallas_call(
      _p2_body,
      grid=(B * S * K // TR,),
      in_specs=[
          pl.BlockSpec((TR, 128), lambda t: (t, 0)),
          pl.BlockSpec((QT, 128), lambda t: (t, 0)),
          pl.BlockSpec((1, 128), lambda t: (0, 0)),
          pl.BlockSpec((1, 128), lambda t: (0, 0)),
          pl.BlockSpec((128, 128), lambda t: (0, 0)),
      ],
      out_specs=[
          pl.BlockSpec((TR, 128), lambda t: (t, 0)),
          pl.BlockSpec((1, 128), lambda t: (0, 0)),
          pl.BlockSpec((1, 128), lambda t: (0, 0)),
      ],
      out_shape=[
          jax.ShapeDtypeStruct((B * S * K, 128), jnp.float32),
          jax.ShapeDtypeStruct((1, 128), jnp.float32),
          jax.ShapeDtypeStruct((1, 128), jnp.float32),
      ],
  )(gath, qp2, a0, c0, w1t)


def _p3_body(y1_ref, a_ref, c_ref, w_ref, y_ref, s1_ref, s2_ref):
  h = jnp.maximum(a_ref[...] * y1_ref[...] + c_ref[...], 0.0)
  y = jnp.dot(h, w_ref[...], preferred_element_type=jnp.float32)
  y_ref[...] = y
  p1 = jnp.sum(y, axis=0, keepdims=True)
  p2 = jnp.sum(y * y, axis=0, keepdims=True)

  @pl.when(pl.program_id(0) == 0)
  def _():
    s1_ref[...] = jnp.zeros_like(s1_ref)
    s2_ref[...] = jnp.zeros_like(s2_ref)

  s1_ref[...] += p1
  s2_ref[...] += p2


def _p3(y1, a1, c1, w2t):
  return pl.pallas_call(
      _p3_body,
      grid=(B * S * K // TR,),
      in_specs=[
          pl.BlockSpec((TR, 128), lambda t: (t, 0)),
          pl.BlockSpec((1, 128), lambda t: (0, 0)),
          pl.BlockSpec((1, 128), lambda t: (0, 0)),
          pl.BlockSpec((128, 256), lambda t: (0, 0)),
      ],
      out_specs=[
          pl.BlockSpec((TR, 256), lambda t: (t, 0)),
          pl.BlockSpec((1, 256), lambda t: (0, 0)),
          pl.BlockSpec((1, 256), lambda t: (0, 0)),
      ],
      out_shape=[
          jax.ShapeDtypeStruct((B * S * K, 256), jnp.float32),
          jax.ShapeDtypeStruct((1, 256), jnp.float32),
          jax.ShapeDtypeStruct((1, 256), jnp.float32),
      ],
  )(y1, a1, c1, w2t)


def _p4_body(y_ref, a_ref, c_ref, o_ref):
  mx = y_ref[:, 0, :]
  mn = y_ref[:, 0, :]
  for k in range(1, K):
    mx = jnp.maximum(mx, y_ref[:, k, :])
    mn = jnp.minimum(mn, y_ref[:, k, :])
  a = a_ref[...]
  m = jnp.where(a >= 0.0, a * mx, a * mn)
  o_ref[...] = jnp.maximum(m + c_ref[...], 0.0)


def _p4(y2_3d, a2, c2):
  return pl.pallas_call(
      _p4_body,
      grid=(B * S // QT,),
      in_specs=[
          pl.BlockSpec((QT, K, 256), lambda t: (t, 0, 0)),
          pl.BlockSpec((1, 256), lambda t: (0, 0)),
          pl.BlockSpec((1, 256), lambda t: (0, 0)),
      ],
      out_specs=pl.BlockSpec((QT, 256), lambda t: (t, 0)),
      out_shape=jax.ShapeDtypeStruct((B * S, 256), jnp.float32),
  )(y2_3d, a2, c2)


def _affine(s1, s2, gamma, beta):
  m = B * S * K
  mean = s1[0] / m
  var = s2[0] / m - mean * mean
  a = gamma / jnp.sqrt(var + 1e-5)
  c = beta - mean * a
  return a[None, :], c[None, :]


def kernel(xyz, features, W0, b0, gamma0, beta0, W1, b1, gamma1, beta1,
           W2, b2, gamma2, beta2):
  xt = jnp.transpose(xyz, (2, 0, 1))
  xp = jnp.concatenate(
      [jnp.transpose(xyz, (0, 2, 1)),
       jnp.zeros((B, 5, N), jnp.float32)], axis=1)

  nxt = _fps(xt)
  new_xyz = jnp.transpose(nxt, (1, 2, 0))
  q = jnp.concatenate(
      [new_xyz, jnp.zeros((B, S, 5), jnp.float32)], axis=2)

  gidx = _knn(q, xp)

  w0f = jnp.transpose(W0[:, 3:])
  w0x = jnp.concatenate(
      [jnp.transpose(W0[:, :3]),
       jnp.zeros((5, 128), jnp.float32)], axis=0)
  t, qp = _p0(features, xp, q, w0f, w0x, b0[None, :])
  table = jnp.reshape(t, (B * N, 128))
  qp2 = jnp.reshape(qp, (B * S, 128))

  gath = _sc_gather(table, jnp.reshape(gidx, (B * S * K,)))

  s1, s2 = _p1(gath, qp2)
  a0, c0 = _affine(s1, s2, gamma0, beta0)
  y1, s1b, s2b = _p2(gath, qp2, a0, c0, jnp.transpose(W1))
  a1, c1 = _affine(s1b, s2b, gamma1, beta1)
  y2, s1c, s2c = _p3(y1, a1, c1, jnp.transpose(W2))
  a2, c2 = _affine(s1c, s2c, gamma2, beta2)
  out = _p4(jnp.reshape(y2, (B * S, K, 256)), a2, c2)

  new_features = jnp.transpose(jnp.reshape(out, (B, S, 256)), (0, 2, 1))
  return new_xyz, new_features
